# baseline (speedup 1.0000x reference)
import functools

import jax
import jax.numpy as jnp
from jax import lax
from jax.experimental import pallas as pl
from jax.experimental.pallas import tpu as pltpu
from jax.experimental.pallas import tpu_sc as plsc

D = 32
K = 20
B = 16384
RPB = 128 // D

NC, NS, L = 2, 16, 16
NW = NC * NS
B_PER_W = B // NW
CHUNK = 32
N_CHUNKS = B_PER_W // CHUNK
G_PER_CHUNK = CHUNK // L
IDX_ROWS = CHUNK * K // 128


def _sc_body(cblk_hbm, ccol_hbm, xblk_hbm, xcol_hbm, in_hbm, out_hbm,
             s_hbm, p_hbm,
             cblk_v, ccol_v, xblk_v, xcol_v, crows, xrows,
             s_stage, p_stage, sem):
    w = lax.axis_index("s") * NC + lax.axis_index("c")
    wbase = w * B_PER_W

    def chunk_body(c, carry):
        base = wbase + c * CHUNK
        pltpu.sync_copy(cblk_hbm.at[pl.ds(base, CHUNK)], cblk_v)
        pltpu.sync_copy(ccol_hbm.at[pl.ds(base, CHUNK)], ccol_v)
        for j in range(IDX_ROWS):
            pltpu.sync_copy(xblk_hbm.at[pl.ds(base * K + j * 128, 128)],
                            xblk_v.at[j])
            pltpu.sync_copy(xcol_hbm.at[pl.ds(base * K + j * 128, 128)],
                            xcol_v.at[pl.ds(j * 128, 128)])
        cps = [pltpu.async_copy(in_hbm.at[cblk_v], crows, sem)]
        for j in range(IDX_ROWS):
            cps.append(pltpu.async_copy(out_hbm.at[xblk_v.at[j]],
                                        xrows.at[pl.ds(j * 128, 128)], sem))
        for cp in cps:
            cp.wait()

        iota = lax.iota(jnp.int32, L)
        for g in range(G_PER_CHUNK):
            cpos = iota + g * L
            ccolv = plsc.load_gather(ccol_v, [cpos])
            cds = [plsc.load_gather(crows, [cpos, ccolv + d])
                   for d in range(D)]
            posbase = iota * K + g * (L * K)
            m = jnp.full((L,), -jnp.inf, jnp.float32)
            ssum = jnp.zeros((L,), jnp.float32)
            tsum = jnp.zeros((L,), jnp.float32)
            for k in range(K):
                posv = posbase + k
                colv = plsc.load_gather(xcol_v, [posv])
                acc = cds[0] * plsc.load_gather(xrows, [posv, colv])
                for d in range(1, D):
                    x = plsc.load_gather(xrows, [posv, colv + d])
                    acc = acc + cds[d] * x
                mn = jnp.maximum(m, acc)
                ssum = ssum * jnp.exp(m - mn) + jnp.exp(acc - mn)
                m = mn
                tsum = tsum + acc
            s_stage[pl.ds(g * L, L)] = ssum
            p_stage[pl.ds(g * L, L)] = K * m - tsum
        pltpu.sync_copy(s_stage, s_hbm.at[pl.ds(base, CHUNK)])
        pltpu.sync_copy(p_stage, p_hbm.at[pl.ds(base, CHUNK)])
        return carry

    lax.fori_loop(0, N_CHUNKS, chunk_body, 0)


_sc_kernel = functools.partial(
    pl.kernel,
    out_type=(jax.ShapeDtypeStruct((B,), jnp.float32),
              jax.ShapeDtypeStruct((B,), jnp.float32)),
    mesh=plsc.VectorSubcoreMesh(core_axis_name="c", subcore_axis_name="s"),
    scratch_types=[
        pltpu.VMEM((CHUNK,), jnp.int32),
        pltpu.VMEM((CHUNK,), jnp.int32),
        pltpu.VMEM((IDX_ROWS, 128), jnp.int32),
        pltpu.VMEM((CHUNK * K,), jnp.int32),
        pltpu.VMEM((CHUNK, 128), jnp.float32),
        pltpu.VMEM((CHUNK * K, 128), jnp.float32),
        pltpu.VMEM((CHUNK,), jnp.float32),
        pltpu.VMEM((CHUNK,), jnp.float32),
        pltpu.SemaphoreType.DMA,
    ],
    compiler_params=pltpu.CompilerParams(needs_layout_passes=False,
                                         use_tc_tiling_on_sc=False),
)(_sc_body)


def _tc_body(s_ref, p_ref, o_ref):
    lse = jnp.log(s_ref[...])
    loss = (jnp.sum(p_ref[...]) + K * jnp.sum(lse)) / (B * K)
    o_ref[...] = loss[None, None]


def kernel(center_ids, context_ids, in_embed, out_embed):
    nblk = in_embed.shape[0] * D // 128
    in2 = in_embed.reshape(nblk, 128)
    out2 = out_embed.reshape(nblk, 128)
    cblk = (center_ids >> 2).astype(jnp.int32)
    ccol = ((center_ids & 3) << 5).astype(jnp.int32)
    xblk = (context_ids >> 2).astype(jnp.int32).reshape(B * K)
    xcol = ((context_ids & 3) << 5).astype(jnp.int32).reshape(B * K)
    s, p = _sc_kernel(cblk, ccol, xblk, xcol, in2, out2)
    loss2d = pl.pallas_call(
        _tc_body,
        out_shape=jax.ShapeDtypeStruct((1, 1), jnp.float32),
    )(s.reshape(128, 128), p.reshape(128, 128))
    return loss2d[0, 0]


# speedup vs baseline: 1.1368x; 1.1368x over previous
import functools

import jax
import jax.numpy as jnp
from jax import lax
from jax.experimental import pallas as pl
from jax.experimental.pallas import tpu as pltpu
from jax.experimental.pallas import tpu_sc as plsc

V = 1000000
D = 32
K = 20
B = 16384

NC, NS, L = 2, 16, 16
NW = NC * NS
B_PER_W = B // NW
CHUNK = 128
N_CHUNKS = B_PER_W // CHUNK
G_PER_CHUNK = CHUNK // L


def _sc_body(cids_hbm, ctxT_hbm, in_hbm, out_hbm, s_hbm, p_hbm,
             cid_v, xblk_v, crows, xrows, s_stage, p_stage, sem):
    w = lax.axis_index("s") * NC + lax.axis_index("c")
    wbase = w * B_PER_W
    iota = lax.iota(jnp.int32, L)

    def chunk_body(c, carry):
        base = wbase + c * CHUNK
        ips = [pltpu.async_copy(cids_hbm.at[pl.ds(base, CHUNK)], cid_v, sem)]
        for k in range(K):
            ips.append(pltpu.async_copy(ctxT_hbm.at[k, pl.ds(base, CHUNK)],
                                        xblk_v.at[k], sem))
        for cp in ips:
            cp.wait()
        cps = [pltpu.async_copy(in_hbm.at[cid_v], crows, sem)]
        for k in range(K):
            cps.append(pltpu.async_copy(out_hbm.at[xblk_v.at[k]],
                                        xrows.at[pl.ds(k * CHUNK, CHUNK)],
                                        sem))
        for cp in cps:
            cp.wait()

        def group_body(g, gcarry):
            cpos = iota + g * L
            cds = [plsc.load_gather(crows, [cpos, jnp.full((L,), d, jnp.int32)])
                   for d in range(D)]
            m = jnp.full((L,), -jnp.inf, jnp.float32)
            ssum = jnp.zeros((L,), jnp.float32)
            tsum = jnp.zeros((L,), jnp.float32)
            for k in range(K):
                posv = iota + (k * CHUNK) + g * L
                acc = cds[0] * plsc.load_gather(
                    xrows, [posv, jnp.zeros((L,), jnp.int32)])
                for d in range(1, D):
                    x = plsc.load_gather(
                        xrows, [posv, jnp.full((L,), d, jnp.int32)])
                    acc = acc + cds[d] * x
                mn = jnp.maximum(m, acc)
                ssum = ssum * jnp.exp(m - mn) + jnp.exp(acc - mn)
                m = mn
                tsum = tsum + acc
            s_stage[pl.ds(g * L, L)] = ssum
            p_stage[pl.ds(g * L, L)] = K * m - tsum
            return gcarry

        lax.fori_loop(0, G_PER_CHUNK, group_body, 0)
        pltpu.sync_copy(s_stage, s_hbm.at[pl.ds(base, CHUNK)])
        pltpu.sync_copy(p_stage, p_hbm.at[pl.ds(base, CHUNK)])
        return carry

    lax.fori_loop(0, N_CHUNKS, chunk_body, 0)


_sc_kernel = functools.partial(
    pl.kernel,
    out_type=(jax.ShapeDtypeStruct((B,), jnp.float32),
              jax.ShapeDtypeStruct((B,), jnp.float32)),
    mesh=plsc.VectorSubcoreMesh(core_axis_name="c", subcore_axis_name="s"),
    scratch_types=[
        pltpu.VMEM((CHUNK,), jnp.int32),
        pltpu.VMEM((K, CHUNK), jnp.int32),
        pltpu.VMEM((CHUNK, D), jnp.float32),
        pltpu.VMEM((CHUNK * K, D), jnp.float32),
        pltpu.VMEM((CHUNK,), jnp.float32),
        pltpu.VMEM((CHUNK,), jnp.float32),
        pltpu.SemaphoreType.DMA,
    ],
    compiler_params=pltpu.CompilerParams(needs_layout_passes=False,
                                         use_tc_tiling_on_sc=False),
)(_sc_body)


def _tc_body(s_ref, p_ref, o_ref):
    lse = jnp.log(s_ref[...])
    loss = (jnp.sum(p_ref[...]) + K * jnp.sum(lse)) / (B * K)
    o_ref[...] = loss[None, None]


def kernel(center_ids, context_ids, in_embed, out_embed):
    ctxT = context_ids.T
    s, p = _sc_kernel(center_ids, ctxT, in_embed, out_embed)
    loss2d = pl.pallas_call(
        _tc_body,
        out_shape=jax.ShapeDtypeStruct((1, 1), jnp.float32),
    )(s.reshape(128, 128), p.reshape(128, 128))
    return loss2d[0, 0]


# speedup vs baseline: 1.7587x; 1.5471x over previous
import functools

import jax
import jax.numpy as jnp
from jax import lax
from jax.experimental import pallas as pl
from jax.experimental.pallas import tpu as pltpu
from jax.experimental.pallas import tpu_sc as plsc

V = 1000000
D = 32
K = 20
B = 16384

NC, NS, L = 2, 16, 16
NW = NC * NS
B_PER_W = B // NW
CHUNK = 128
N_CHUNKS = B_PER_W // CHUNK
G_PER_CHUNK = CHUNK // L


def _sc_body(cids_hbm, ctxT_hbm, in_hbm, out_hbm, s_hbm, p_hbm,
             cid_v, xblk_v, crows, xrows, s_stage, p_stage, sem):
    w = lax.axis_index("s") * NC + lax.axis_index("c")
    wbase = w * B_PER_W
    iota = lax.iota(jnp.int32, L)

    def chunk_body(c, carry):
        base = wbase + c * CHUNK
        ips = [pltpu.async_copy(cids_hbm.at[pl.ds(base, CHUNK)], cid_v, sem)]
        for k in range(K):
            ips.append(pltpu.async_copy(ctxT_hbm.at[k, pl.ds(base, CHUNK)],
                                        xblk_v.at[k], sem))
        for cp in ips:
            cp.wait()
        cps = [pltpu.async_copy(in_hbm.at[cid_v], crows, sem)]
        for k in range(K):
            cps.append(pltpu.async_copy(out_hbm.at[xblk_v.at[k]],
                                        xrows.at[pl.ds(k * CHUNK, CHUNK)],
                                        sem))
        for cp in cps:
            cp.wait()

        def group_body(g, gcarry):
            cpos = iota + g * L
            cds = [plsc.load_gather(crows, [cpos, jnp.full((L,), d, jnp.int32)])
                   for d in range(D)]
            m = jnp.full((L,), -jnp.inf, jnp.float32)
            ssum = jnp.zeros((L,), jnp.float32)
            tsum = jnp.zeros((L,), jnp.float32)
            for k in range(K):
                posv = iota + (k * CHUNK) + g * L
                acc = cds[0] * plsc.load_gather(
                    xrows, [posv, jnp.zeros((L,), jnp.int32)])
                for d in range(1, D):
                    x = plsc.load_gather(
                        xrows, [posv, jnp.full((L,), d, jnp.int32)])
                    acc = acc + cds[d] * x
                mn = jnp.maximum(m, acc)
                ssum = ssum * jnp.exp(m - mn) + jnp.exp(acc - mn)
                m = mn
                tsum = tsum + acc
            s_stage[pl.ds(g * L, L)] = ssum
            p_stage[pl.ds(g * L, L)] = K * m - tsum
            return gcarry

        lax.fori_loop(0, G_PER_CHUNK, group_body, 0)
        pltpu.sync_copy(s_stage, s_hbm.at[pl.ds(base, CHUNK)])
        pltpu.sync_copy(p_stage, p_hbm.at[pl.ds(base, CHUNK)])
        return carry

    lax.fori_loop(0, N_CHUNKS, chunk_body, 0)


_sc_kernel = functools.partial(
    pl.kernel,
    out_type=(jax.ShapeDtypeStruct((B,), jnp.float32),
              jax.ShapeDtypeStruct((B,), jnp.float32)),
    mesh=plsc.VectorSubcoreMesh(core_axis_name="c", subcore_axis_name="s"),
    scratch_types=[
        pltpu.VMEM((CHUNK,), jnp.int32),
        pltpu.VMEM((K, CHUNK), jnp.int32),
        pltpu.VMEM((CHUNK, D), jnp.float32),
        pltpu.VMEM((CHUNK * K, D), jnp.float32),
        pltpu.VMEM((CHUNK,), jnp.float32),
        pltpu.VMEM((CHUNK,), jnp.float32),
        pltpu.SemaphoreType.DMA,
    ],
    compiler_params=pltpu.CompilerParams(needs_layout_passes=False,
                                         use_tc_tiling_on_sc=False),
)(_sc_body)


def _tc_body(s_ref, p_ref, o_ref):
    lse = jnp.log(s_ref[...])
    loss = (jnp.sum(p_ref[...]) + K * jnp.sum(lse)) / (B * K)
    o_ref[...] = loss[None, None]


_TRBC = 2048
_TRG = -(-V // (4 * _TRBC))
_VP = 4 * _TRBC * _TRG


def _tr_body(x_ref, o_ref):
    x = x_ref[...]
    o_ref[...] = jnp.concatenate(
        [x[:, j * _TRBC:(j + 1) * _TRBC].T for j in range(4)], axis=1)


def _transpose_table(tT):
    return pl.pallas_call(
        _tr_body,
        grid=(_TRG,),
        in_specs=[pl.BlockSpec((D, 4 * _TRBC), lambda c: (0, c))],
        out_specs=pl.BlockSpec((_TRBC, 128), lambda c: (c, 0)),
        out_shape=jax.ShapeDtypeStruct((_VP * D // 128, 128), jnp.float32),
    )(tT)


def _rho(ids):
    g = ids // (4 * _TRBC)
    r = ids % (4 * _TRBC)
    return g * (4 * _TRBC) + (r % _TRBC) * 4 + r // _TRBC


def kernel(center_ids, context_ids, in_embed, out_embed):
    ctxT = _rho(context_ids).T
    in_t = _transpose_table(in_embed.T).reshape(_VP, D)
    out_t = _transpose_table(out_embed.T).reshape(_VP, D)
    s, p = _sc_kernel(_rho(center_ids), ctxT, in_t, out_t)
    loss2d = pl.pallas_call(
        _tc_body,
        out_shape=jax.ShapeDtypeStruct((1, 1), jnp.float32),
    )(s.reshape(128, 128), p.reshape(128, 128))
    return loss2d[0, 0]


# speedup vs baseline: 2.6154x; 1.4871x over previous
import functools

import jax
import jax.numpy as jnp
from jax import lax
from jax.experimental import pallas as pl
from jax.experimental.pallas import tpu as pltpu
from jax.experimental.pallas import tpu_sc as plsc

V = 1000000
D = 32
K = 20
B = 16384

NC, NS, L = 2, 16, 16
NW = NC * NS
B_PER_W = B // NW
CHUNK = 128
N_CHUNKS = B_PER_W // CHUNK
G_PER_CHUNK = CHUNK // L


def _sc_body(cids_hbm, ctxT_hbm, in_hbm, out_hbm, s_hbm, p_hbm,
             cid_v, xblk_v, crows, xrows, s_stage, p_stage, sem):
    w = lax.axis_index("s") * NC + lax.axis_index("c")
    wbase = w * B_PER_W
    iota = lax.iota(jnp.int32, L)

    def chunk_body(c, carry):
        base = wbase + c * CHUNK
        ips = [pltpu.async_copy(cids_hbm.at[pl.ds(base, CHUNK)], cid_v, sem)]
        for k in range(K):
            ips.append(pltpu.async_copy(ctxT_hbm.at[k, pl.ds(base, CHUNK)],
                                        xblk_v.at[k], sem))
        for cp in ips:
            cp.wait()
        cps = [pltpu.async_copy(in_hbm.at[cid_v], crows, sem)]
        for k in range(K):
            cps.append(pltpu.async_copy(out_hbm.at[xblk_v.at[k]],
                                        xrows.at[pl.ds(k * CHUNK, CHUNK)],
                                        sem))
        for cp in cps:
            cp.wait()

        def group_body(g, gcarry):
            cpos = iota + g * L
            cds = [plsc.load_gather(crows, [cpos, jnp.full((L,), d, jnp.int32)])
                   for d in range(D)]
            m = jnp.full((L,), -jnp.inf, jnp.float32)
            ssum = jnp.zeros((L,), jnp.float32)
            tsum = jnp.zeros((L,), jnp.float32)
            for k in range(K):
                posv = iota + (k * CHUNK) + g * L
                acc = cds[0] * plsc.load_gather(
                    xrows, [posv, jnp.zeros((L,), jnp.int32)])
                for d in range(1, D):
                    x = plsc.load_gather(
                        xrows, [posv, jnp.full((L,), d, jnp.int32)])
                    acc = acc + cds[d] * x
                mn = jnp.maximum(m, acc)
                ssum = ssum * jnp.exp(m - mn) + jnp.exp(acc - mn)
                m = mn
                tsum = tsum + acc
            s_stage[pl.ds(g * L, L)] = ssum
            p_stage[pl.ds(g * L, L)] = K * m - tsum
            return gcarry

        lax.fori_loop(0, G_PER_CHUNK, group_body, 0)
        pltpu.sync_copy(s_stage, s_hbm.at[pl.ds(base, CHUNK)])
        pltpu.sync_copy(p_stage, p_hbm.at[pl.ds(base, CHUNK)])
        return carry

    lax.fori_loop(0, N_CHUNKS, chunk_body, 0)


_sc_kernel = functools.partial(
    pl.kernel,
    out_type=(jax.ShapeDtypeStruct((B,), jnp.float32),
              jax.ShapeDtypeStruct((B,), jnp.float32)),
    mesh=plsc.VectorSubcoreMesh(core_axis_name="c", subcore_axis_name="s"),
    scratch_types=[
        pltpu.VMEM((CHUNK,), jnp.int32),
        pltpu.VMEM((K, CHUNK), jnp.int32),
        pltpu.VMEM((CHUNK, D), jnp.float32),
        pltpu.VMEM((CHUNK * K, D), jnp.float32),
        pltpu.VMEM((CHUNK,), jnp.float32),
        pltpu.VMEM((CHUNK,), jnp.float32),
        pltpu.SemaphoreType.DMA,
    ],
    compiler_params=pltpu.CompilerParams(needs_layout_passes=False,
                                         use_tc_tiling_on_sc=False),
)(_sc_body)


def _tc_body(s_ref, p_ref, o_ref):
    lse = jnp.log(s_ref[...])
    loss = (jnp.sum(p_ref[...]) + K * jnp.sum(lse)) / (B * K)
    o_ref[...] = loss[None, None]


_TRBC = 2048
_TRG = -(-V // (4 * _TRBC))
_VP = 4 * _TRBC * _TRG


def _tr_body(x_ref, o_ref):
    x = x_ref[...]
    o_ref[...] = jnp.concatenate(
        [x[:, j * _TRBC:(j + 1) * _TRBC] for j in range(4)], axis=0).T


def _transpose_table(tT):
    return pl.pallas_call(
        _tr_body,
        grid=(_TRG,),
        in_specs=[pl.BlockSpec((D, 4 * _TRBC), lambda c: (0, c))],
        out_specs=pl.BlockSpec((_TRBC, 128), lambda c: (c, 0)),
        out_shape=jax.ShapeDtypeStruct((_VP * D // 128, 128), jnp.float32),
    )(tT)


def _rho(ids):
    g = ids // (4 * _TRBC)
    r = ids % (4 * _TRBC)
    return g * (4 * _TRBC) + (r % _TRBC) * 4 + r // _TRBC


def kernel(center_ids, context_ids, in_embed, out_embed):
    ctxT = _rho(context_ids).T
    in_t = _transpose_table(in_embed.T).reshape(_VP, D)
    out_t = _transpose_table(out_embed.T).reshape(_VP, D)
    s, p = _sc_kernel(_rho(center_ids), ctxT, in_t, out_t)
    loss2d = pl.pallas_call(
        _tc_body,
        out_shape=jax.ShapeDtypeStruct((1, 1), jnp.float32),
    )(s.reshape(128, 128), p.reshape(128, 128))
    return loss2d[0, 0]


# speedup vs baseline: 3.4103x; 1.3039x over previous
import functools

import jax
import jax.numpy as jnp
from jax import lax
from jax.experimental import pallas as pl
from jax.experimental.pallas import tpu as pltpu
from jax.experimental.pallas import tpu_sc as plsc

V = 1000000
D = 32
K = 20
B = 16384

NC, NS, L = 2, 16, 16
NW = NC * NS
B_PER_W = B // NW
CHUNK = 128
N_CHUNKS = B_PER_W // CHUNK
G_PER_CHUNK = CHUNK // L


def _sc_body(cids_hbm, ctxT_hbm, in_hbm, out_hbm, s_hbm, p_hbm,
             cid_v, xblk_v, crows, xrows, s_stage, p_stage, sem):
    w = lax.axis_index("s") * NC + lax.axis_index("c")
    wbase = w * B_PER_W
    iota = lax.iota(jnp.int32, L)

    def chunk_body(c, carry):
        base = wbase + c * CHUNK
        ips = [pltpu.async_copy(cids_hbm.at[pl.ds(base, CHUNK)], cid_v, sem)]
        for k in range(K):
            ips.append(pltpu.async_copy(ctxT_hbm.at[k, pl.ds(base, CHUNK)],
                                        xblk_v.at[k], sem))
        for cp in ips:
            cp.wait()
        cps = [pltpu.async_copy(in_hbm.at[cid_v], crows, sem)]
        for k in range(K):
            cps.append(pltpu.async_copy(out_hbm.at[xblk_v.at[k]],
                                        xrows.at[pl.ds(k * CHUNK, CHUNK)],
                                        sem))
        for cp in cps:
            cp.wait()

        def group_body(g, gcarry):
            cpos = iota + g * L
            rcols = [jnp.bitwise_and(iota + d, D - 1) for d in range(D)]
            cds = [plsc.load_gather(crows, [cpos, rcols[d]])
                   for d in range(D)]
            m = jnp.full((L,), -jnp.inf, jnp.float32)
            ssum = jnp.zeros((L,), jnp.float32)
            tsum = jnp.zeros((L,), jnp.float32)
            for k in range(K):
                posv = iota + (k * CHUNK) + g * L
                acc = cds[0] * plsc.load_gather(xrows, [posv, rcols[0]])
                for d in range(1, D):
                    x = plsc.load_gather(xrows, [posv, rcols[d]])
                    acc = acc + cds[d] * x
                mn = jnp.maximum(m, acc)
                ssum = ssum * jnp.exp(m - mn) + jnp.exp(acc - mn)
                m = mn
                tsum = tsum + acc
            s_stage[pl.ds(g * L, L)] = ssum
            p_stage[pl.ds(g * L, L)] = K * m - tsum
            return gcarry

        lax.fori_loop(0, G_PER_CHUNK, group_body, 0)
        pltpu.sync_copy(s_stage, s_hbm.at[pl.ds(base, CHUNK)])
        pltpu.sync_copy(p_stage, p_hbm.at[pl.ds(base, CHUNK)])
        return carry

    lax.fori_loop(0, N_CHUNKS, chunk_body, 0)


_sc_kernel = functools.partial(
    pl.kernel,
    out_type=(jax.ShapeDtypeStruct((B,), jnp.float32),
              jax.ShapeDtypeStruct((B,), jnp.float32)),
    mesh=plsc.VectorSubcoreMesh(core_axis_name="c", subcore_axis_name="s"),
    scratch_types=[
        pltpu.VMEM((CHUNK,), jnp.int32),
        pltpu.VMEM((K, CHUNK), jnp.int32),
        pltpu.VMEM((CHUNK, D), jnp.float32),
        pltpu.VMEM((CHUNK * K, D), jnp.float32),
        pltpu.VMEM((CHUNK,), jnp.float32),
        pltpu.VMEM((CHUNK,), jnp.float32),
        pltpu.SemaphoreType.DMA,
    ],
    compiler_params=pltpu.CompilerParams(needs_layout_passes=False,
                                         use_tc_tiling_on_sc=False),
)(_sc_body)


def _tc_body(s_ref, p_ref, o_ref):
    lse = jnp.log(s_ref[...])
    loss = (jnp.sum(p_ref[...]) + K * jnp.sum(lse)) / (B * K)
    o_ref[...] = loss[None, None]


_TRBC = 2048
_TRG = -(-V // (4 * _TRBC))
_VP = 4 * _TRBC * _TRG


def _tr_body(x_ref, o_ref):
    x = x_ref[...]
    o_ref[...] = jnp.concatenate(
        [x[:, j * _TRBC:(j + 1) * _TRBC] for j in range(4)], axis=0).T


def _transpose_table(tT):
    return pl.pallas_call(
        _tr_body,
        grid=(_TRG,),
        in_specs=[pl.BlockSpec((D, 4 * _TRBC), lambda c: (0, c))],
        out_specs=pl.BlockSpec((_TRBC, 128), lambda c: (c, 0)),
        out_shape=jax.ShapeDtypeStruct((_VP * D // 128, 128), jnp.float32),
    )(tT)


def _rho(ids):
    g = ids // (4 * _TRBC)
    r = ids % (4 * _TRBC)
    return g * (4 * _TRBC) + (r % _TRBC) * 4 + r // _TRBC


def kernel(center_ids, context_ids, in_embed, out_embed):
    ctxT = _rho(context_ids).T
    in_t = _transpose_table(in_embed.T).reshape(_VP, D)
    out_t = _transpose_table(out_embed.T).reshape(_VP, D)
    s, p = _sc_kernel(_rho(center_ids), ctxT, in_t, out_t)
    loss2d = pl.pallas_call(
        _tc_body,
        out_shape=jax.ShapeDtypeStruct((1, 1), jnp.float32),
    )(s.reshape(128, 128), p.reshape(128, 128))
    return loss2d[0, 0]


# speedup vs baseline: 4.0891x; 1.1991x over previous
import functools

import jax
import jax.numpy as jnp
from jax import lax
from jax.experimental import pallas as pl
from jax.experimental.pallas import tpu as pltpu
from jax.experimental.pallas import tpu_sc as plsc

V = 1000000
D = 32
K = 20
B = 16384

NC, NS, L = 2, 16, 16
NW = NC * NS
B_PER_W = B // NW
CHUNK = 64
N_CHUNKS = B_PER_W // CHUNK
G_PER_CHUNK = CHUNK // L


def _sc_body(cids_hbm, ctxT_hbm, in_hbm, out_hbm, s_hbm, p_hbm,
             cid_v, xblk_v, crows, xrows, s_stage, p_stage,
             semi0, semi1, semg0, semg1):
    w = lax.axis_index("s") * NC + lax.axis_index("c")
    wbase = w * B_PER_W
    iota = lax.iota(jnp.int32, L)
    semi = (semi0, semi1)
    semg = (semg0, semg1)

    def ids_descs(c, slot):
        base = wbase + (c % N_CHUNKS) * CHUNK
        ds = [pltpu.make_async_copy(cids_hbm.at[pl.ds(base, CHUNK)],
                                    cid_v.at[slot], semi[slot])]
        for k in range(K):
            ds.append(pltpu.make_async_copy(
                ctxT_hbm.at[k, pl.ds(base, CHUNK)],
                xblk_v.at[slot, k], semi[slot]))
        return ds

    def gather_descs(slot):
        ds = [pltpu.make_async_copy(in_hbm.at[cid_v.at[slot]],
                                    crows.at[slot], semg[slot])]
        for k in range(K):
            ds.append(pltpu.make_async_copy(
                out_hbm.at[xblk_v.at[slot, k]],
                xrows.at[slot, pl.ds(k * CHUNK, CHUNK)], semg[slot]))
        return ds

    def start(descs):
        for cp in descs:
            cp.start()

    def wait(descs):
        for cp in descs:
            cp.wait()

    def compute(c, slot):
        rcols = [jnp.bitwise_and(iota + d, D - 1) for d in range(D)]

        def group_body(g, gcarry):
            cpos = iota + g * L
            cds = [plsc.load_gather(crows.at[slot], [cpos, rcols[d]])
                   for d in range(D)]
            m = jnp.full((L,), -jnp.inf, jnp.float32)
            ssum = jnp.zeros((L,), jnp.float32)
            tsum = jnp.zeros((L,), jnp.float32)
            for k in range(K):
                posv = iota + (k * CHUNK) + g * L
                acc = cds[0] * plsc.load_gather(xrows.at[slot],
                                                [posv, rcols[0]])
                for d in range(1, D):
                    x = plsc.load_gather(xrows.at[slot], [posv, rcols[d]])
                    acc = acc + cds[d] * x
                mn = jnp.maximum(m, acc)
                ssum = ssum * jnp.exp(m - mn) + jnp.exp(acc - mn)
                m = mn
                tsum = tsum + acc
            off = c * CHUNK + g * L
            s_stage[pl.ds(off, L)] = ssum
            p_stage[pl.ds(off, L)] = K * m - tsum
            return gcarry

        lax.fori_loop(0, G_PER_CHUNK, group_body, 0)

    start(ids_descs(0, 0))
    wait(ids_descs(0, 0))
    start(gather_descs(0))
    start(ids_descs(1, 1))

    def pipe_body(t, carry):
        c0 = 2 * t
        wait(ids_descs(c0 + 1, 1))
        start(gather_descs(1))
        wait(gather_descs(0))
        start(ids_descs(c0 + 2, 0))
        compute(c0, 0)
        wait(ids_descs(c0 + 2, 0))
        start(gather_descs(0))
        wait(gather_descs(1))
        start(ids_descs(c0 + 3, 1))
        compute(c0 + 1, 1)
        return carry

    lax.fori_loop(0, N_CHUNKS // 2, pipe_body, 0)
    wait(gather_descs(0))
    wait(ids_descs(0, 1))
    pltpu.sync_copy(s_stage, s_hbm.at[pl.ds(wbase, B_PER_W)])
    pltpu.sync_copy(p_stage, p_hbm.at[pl.ds(wbase, B_PER_W)])


_sc_kernel = functools.partial(
    pl.kernel,
    out_type=(jax.ShapeDtypeStruct((B,), jnp.float32),
              jax.ShapeDtypeStruct((B,), jnp.float32)),
    mesh=plsc.VectorSubcoreMesh(core_axis_name="c", subcore_axis_name="s"),
    scratch_types=[
        pltpu.VMEM((2, CHUNK), jnp.int32),
        pltpu.VMEM((2, K, CHUNK), jnp.int32),
        pltpu.VMEM((2, CHUNK, D), jnp.float32),
        pltpu.VMEM((2, CHUNK * K, D), jnp.float32),
        pltpu.VMEM((B_PER_W,), jnp.float32),
        pltpu.VMEM((B_PER_W,), jnp.float32),
        pltpu.SemaphoreType.DMA,
        pltpu.SemaphoreType.DMA,
        pltpu.SemaphoreType.DMA,
        pltpu.SemaphoreType.DMA,
    ],
    compiler_params=pltpu.CompilerParams(needs_layout_passes=False,
                                         use_tc_tiling_on_sc=False),
)(_sc_body)


def _tc_body(s_ref, p_ref, o_ref):
    lse = jnp.log(s_ref[...])
    loss = (jnp.sum(p_ref[...]) + K * jnp.sum(lse)) / (B * K)
    o_ref[...] = loss[None, None]


_TRBC = 2048
_TRG = -(-V // (4 * _TRBC))
_VP = 4 * _TRBC * _TRG


def _tr_body(x_ref, y_ref, ox_ref, oy_ref):
    x = x_ref[...]
    ox_ref[...] = jnp.concatenate(
        [x[:, j * _TRBC:(j + 1) * _TRBC] for j in range(4)], axis=0).T
    y = y_ref[...]
    oy_ref[...] = jnp.concatenate(
        [y[:, j * _TRBC:(j + 1) * _TRBC] for j in range(4)], axis=0).T


def _transpose_tables(aT, bT):
    spec_in = pl.BlockSpec((D, 4 * _TRBC), lambda c: (0, c))
    spec_out = pl.BlockSpec((_TRBC, 128), lambda c: (c, 0))
    shape = jax.ShapeDtypeStruct((_VP * D // 128, 128), jnp.float32)
    return pl.pallas_call(
        _tr_body,
        grid=(_TRG,),
        in_specs=[spec_in, spec_in],
        out_specs=[spec_out, spec_out],
        out_shape=[shape, shape],
    )(aT, bT)


def _rho(ids):
    g = ids // (4 * _TRBC)
    r = ids % (4 * _TRBC)
    return g * (4 * _TRBC) + (r % _TRBC) * 4 + r // _TRBC


def kernel(center_ids, context_ids, in_embed, out_embed):
    ctxT = _rho(context_ids).T
    in_t, out_t = _transpose_tables(in_embed.T, out_embed.T)
    in_t = in_t.reshape(_VP, D)
    out_t = out_t.reshape(_VP, D)
    s, p = _sc_kernel(_rho(center_ids), ctxT, in_t, out_t)
    loss2d = pl.pallas_call(
        _tc_body,
        out_shape=jax.ShapeDtypeStruct((1, 1), jnp.float32),
    )(s.reshape(128, 128), p.reshape(128, 128))
    return loss2d[0, 0]


# speedup vs baseline: 4.1524x; 1.0155x over previous
import functools

import jax
import jax.numpy as jnp
from jax import lax
from jax.experimental import pallas as pl
from jax.experimental.pallas import tpu as pltpu
from jax.experimental.pallas import tpu_sc as plsc

V = 1000000
D = 32
K = 20
B = 16384

NC, NS, L = 2, 16, 16
NW = NC * NS
B_PER_W = B // NW
CHUNK = 64
N_CHUNKS = B_PER_W // CHUNK
G_PER_CHUNK = CHUNK // L


def _sc_body(cids_hbm, ctxT_hbm, in_hbm, out_hbm, s_hbm, p_hbm,
             cid_v, xblk_v, crows, xrows, s_stage, p_stage,
             semi0, semi1, semg0, semg1):
    w = lax.axis_index("s") * NC + lax.axis_index("c")
    wbase = w * B_PER_W
    iota = lax.iota(jnp.int32, L)
    semi = (semi0, semi1)
    semg = (semg0, semg1)

    def ids_descs(c, slot):
        base = wbase + (c % N_CHUNKS) * CHUNK
        ds = [pltpu.make_async_copy(cids_hbm.at[pl.ds(base, CHUNK)],
                                    cid_v.at[slot], semi[slot])]
        for k in range(K):
            ds.append(pltpu.make_async_copy(
                ctxT_hbm.at[k, pl.ds(base, CHUNK)],
                xblk_v.at[slot, pl.ds(k * CHUNK, CHUNK)], semi[slot]))
        return ds

    def gather_descs(slot):
        return [
            pltpu.make_async_copy(in_hbm.at[cid_v.at[slot]],
                                  crows.at[slot], semg[slot]),
            pltpu.make_async_copy(out_hbm.at[xblk_v.at[slot]],
                                  xrows.at[slot], semg[slot]),
        ]

    def start(descs):
        for cp in descs:
            cp.start()

    def wait(descs):
        for cp in descs:
            cp.wait()

    def compute(c, slot):
        rcols = [jnp.bitwise_and(iota + d, D - 1) for d in range(D)]

        def group_body(g, gcarry):
            cpos = iota + g * L
            cds = [plsc.load_gather(crows.at[slot], [cpos, rcols[d]])
                   for d in range(D)]
            m = jnp.full((L,), -jnp.inf, jnp.float32)
            ssum = jnp.zeros((L,), jnp.float32)
            tsum = jnp.zeros((L,), jnp.float32)
            for k in range(K):
                posv = iota + (k * CHUNK) + g * L
                acc = cds[0] * plsc.load_gather(xrows.at[slot],
                                                [posv, rcols[0]])
                for d in range(1, D):
                    x = plsc.load_gather(xrows.at[slot], [posv, rcols[d]])
                    acc = acc + cds[d] * x
                mn = jnp.maximum(m, acc)
                ssum = ssum * jnp.exp(m - mn) + jnp.exp(acc - mn)
                m = mn
                tsum = tsum + acc
            off = c * CHUNK + g * L
            s_stage[pl.ds(off, L)] = ssum
            p_stage[pl.ds(off, L)] = K * m - tsum
            return gcarry

        lax.fori_loop(0, G_PER_CHUNK, group_body, 0)

    start(ids_descs(0, 0))
    wait(ids_descs(0, 0))
    start(gather_descs(0))
    start(ids_descs(1, 1))

    def pipe_body(t, carry):
        c0 = 2 * t
        wait(ids_descs(c0 + 1, 1))
        start(gather_descs(1))
        wait(gather_descs(0))
        start(ids_descs(c0 + 2, 0))
        compute(c0, 0)
        wait(ids_descs(c0 + 2, 0))
        start(gather_descs(0))
        wait(gather_descs(1))
        start(ids_descs(c0 + 3, 1))
        compute(c0 + 1, 1)
        return carry

    lax.fori_loop(0, N_CHUNKS // 2, pipe_body, 0)
    wait(gather_descs(0))
    wait(ids_descs(0, 1))
    pltpu.sync_copy(s_stage, s_hbm.at[pl.ds(wbase, B_PER_W)])
    pltpu.sync_copy(p_stage, p_hbm.at[pl.ds(wbase, B_PER_W)])


_sc_kernel = functools.partial(
    pl.kernel,
    out_type=(jax.ShapeDtypeStruct((B,), jnp.float32),
              jax.ShapeDtypeStruct((B,), jnp.float32)),
    mesh=plsc.VectorSubcoreMesh(core_axis_name="c", subcore_axis_name="s"),
    scratch_types=[
        pltpu.VMEM((2, CHUNK), jnp.int32),
        pltpu.VMEM((2, K * CHUNK), jnp.int32),
        pltpu.VMEM((2, CHUNK, D), jnp.float32),
        pltpu.VMEM((2, CHUNK * K, D), jnp.float32),
        pltpu.VMEM((B_PER_W,), jnp.float32),
        pltpu.VMEM((B_PER_W,), jnp.float32),
        pltpu.SemaphoreType.DMA,
        pltpu.SemaphoreType.DMA,
        pltpu.SemaphoreType.DMA,
        pltpu.SemaphoreType.DMA,
    ],
    compiler_params=pltpu.CompilerParams(needs_layout_passes=False,
                                         use_tc_tiling_on_sc=False),
)(_sc_body)


def _tc_body(s_ref, p_ref, o_ref):
    lse = jnp.log(s_ref[...])
    loss = (jnp.sum(p_ref[...]) + K * jnp.sum(lse)) / (B * K)
    o_ref[...] = loss[None, None]


_TRBC = 2048
_TRG = -(-V // (4 * _TRBC))
_VP = 4 * _TRBC * _TRG


def _tr_body(x_ref, y_ref, ox_ref, oy_ref):
    x = x_ref[...]
    ox_ref[...] = jnp.concatenate(
        [x[:, j * _TRBC:(j + 1) * _TRBC] for j in range(4)], axis=0).T
    y = y_ref[...]
    oy_ref[...] = jnp.concatenate(
        [y[:, j * _TRBC:(j + 1) * _TRBC] for j in range(4)], axis=0).T


def _transpose_tables(aT, bT):
    spec_in = pl.BlockSpec((D, 4 * _TRBC), lambda c: (0, c))
    spec_out = pl.BlockSpec((_TRBC, 128), lambda c: (c, 0))
    shape = jax.ShapeDtypeStruct((_VP * D // 128, 128), jnp.float32)
    return pl.pallas_call(
        _tr_body,
        grid=(_TRG,),
        in_specs=[spec_in, spec_in],
        out_specs=[spec_out, spec_out],
        out_shape=[shape, shape],
    )(aT, bT)


def _rho(ids):
    g = ids // (4 * _TRBC)
    r = ids % (4 * _TRBC)
    return g * (4 * _TRBC) + (r % _TRBC) * 4 + r // _TRBC


def kernel(center_ids, context_ids, in_embed, out_embed):
    ctxT = _rho(context_ids).T
    in_t, out_t = _transpose_tables(in_embed.T, out_embed.T)
    in_t = in_t.reshape(_VP, D)
    out_t = out_t.reshape(_VP, D)
    s, p = _sc_kernel(_rho(center_ids), ctxT, in_t, out_t)
    loss2d = pl.pallas_call(
        _tc_body,
        out_shape=jax.ShapeDtypeStruct((1, 1), jnp.float32),
    )(s.reshape(128, 128), p.reshape(128, 128))
    return loss2d[0, 0]


# speedup vs baseline: 4.6197x; 1.1125x over previous
import functools

import jax
import jax.numpy as jnp
from jax import lax
from jax.experimental import pallas as pl
from jax.experimental.pallas import tpu as pltpu
from jax.experimental.pallas import tpu_sc as plsc

V = 1000000
D = 32
K = 20
B = 16384

NC, NS, L = 2, 16, 16
NW = NC * NS
B_PER_W = B // NW
CHUNK = 64
N_CHUNKS = B_PER_W // CHUNK
G_PER_CHUNK = CHUNK // L


def _sc_body(cids_hbm, ctxT_hbm, in_hbm, out_hbm, s_hbm, p_hbm,
             cid_v, xblk_v, crows, xrows, s_stage, p_stage,
             semi0, semi1, semg0, semg1):
    w = lax.axis_index("s") * NC + lax.axis_index("c")
    wbase = w * B_PER_W
    iota = lax.iota(jnp.int32, L)
    semi = (semi0, semi1)
    semg = (semg0, semg1)

    def ids_descs(c, slot):
        base = wbase + (c % N_CHUNKS) * CHUNK
        ds = [pltpu.make_async_copy(cids_hbm.at[pl.ds(base, CHUNK)],
                                    cid_v.at[slot], semi[slot])]
        for k in range(K):
            ds.append(pltpu.make_async_copy(
                ctxT_hbm.at[k, pl.ds(base, CHUNK)],
                xblk_v.at[slot, pl.ds(k * CHUNK, CHUNK)], semi[slot]))
        return ds

    def gather_descs(slot):
        return [
            pltpu.make_async_copy(in_hbm.at[cid_v.at[slot]],
                                  crows.at[slot], semg[slot]),
            pltpu.make_async_copy(out_hbm.at[xblk_v.at[slot]],
                                  xrows.at[slot], semg[slot]),
        ]

    def start(descs):
        for cp in descs:
            cp.start()

    def wait(descs):
        for cp in descs:
            cp.wait()

    def compute(c, slot):
        rcols = [jnp.bitwise_and(iota + d, D - 1) for d in range(D)]

        def group_body(g, gcarry):
            cpos = iota + g * L
            cds = [plsc.load_gather(crows.at[slot], [cpos, rcols[d]])
                   for d in range(D)]
            m = jnp.full((L,), -jnp.inf, jnp.float32)
            ssum = jnp.zeros((L,), jnp.float32)
            tsum = jnp.zeros((L,), jnp.float32)
            for k in range(K):
                posv = iota + (k * CHUNK) + g * L
                acc = cds[0] * plsc.load_gather(xrows.at[slot],
                                                [posv, rcols[0]])
                for d in range(1, D):
                    x = plsc.load_gather(xrows.at[slot], [posv, rcols[d]])
                    acc = acc + cds[d] * x
                mn = jnp.maximum(m, acc)
                ssum = ssum * jnp.exp(m - mn) + jnp.exp(acc - mn)
                m = mn
                tsum = tsum + acc
            off = c * CHUNK + g * L
            s_stage[pl.ds(off, L)] = ssum
            p_stage[pl.ds(off, L)] = K * m - tsum
            return gcarry

        lax.fori_loop(0, G_PER_CHUNK, group_body, 0)

    start(ids_descs(0, 0))
    wait(ids_descs(0, 0))
    start(gather_descs(0))
    start(ids_descs(1, 1))

    def pipe_body(t, carry):
        c0 = 2 * t
        wait(ids_descs(c0 + 1, 1))
        start(gather_descs(1))
        wait(gather_descs(0))
        start(ids_descs(c0 + 2, 0))
        compute(c0, 0)
        wait(ids_descs(c0 + 2, 0))
        start(gather_descs(0))
        wait(gather_descs(1))
        start(ids_descs(c0 + 3, 1))
        compute(c0 + 1, 1)
        return carry

    lax.fori_loop(0, N_CHUNKS // 2, pipe_body, 0)
    wait(gather_descs(0))
    wait(ids_descs(0, 1))
    pltpu.sync_copy(s_stage, s_hbm.at[pl.ds(wbase, B_PER_W)])
    pltpu.sync_copy(p_stage, p_hbm.at[pl.ds(wbase, B_PER_W)])


_sc_kernel = functools.partial(
    pl.kernel,
    out_type=(jax.ShapeDtypeStruct((B,), jnp.float32),
              jax.ShapeDtypeStruct((B,), jnp.float32)),
    mesh=plsc.VectorSubcoreMesh(core_axis_name="c", subcore_axis_name="s"),
    scratch_types=[
        pltpu.VMEM((2, CHUNK), jnp.int32),
        pltpu.VMEM((2, K * CHUNK), jnp.int32),
        pltpu.VMEM((2, CHUNK, D), jnp.float32),
        pltpu.VMEM((2, CHUNK * K, D), jnp.float32),
        pltpu.VMEM((B_PER_W,), jnp.float32),
        pltpu.VMEM((B_PER_W,), jnp.float32),
        pltpu.SemaphoreType.DMA,
        pltpu.SemaphoreType.DMA,
        pltpu.SemaphoreType.DMA,
        pltpu.SemaphoreType.DMA,
    ],
    compiler_params=pltpu.CompilerParams(needs_layout_passes=False,
                                         use_tc_tiling_on_sc=False),
)(_sc_body)


def _tc_body(s_ref, p_ref, o_ref):
    lse = jnp.log(s_ref[...])
    loss = (jnp.sum(p_ref[...]) + K * jnp.sum(lse)) / (B * K)
    o_ref[...] = loss[None, None]


_TRBC = 4096
_TRG = -(-V // (4 * _TRBC))
_VP = 4 * _TRBC * _TRG


def _tr_body(x_ref, y_ref, ox_ref, oy_ref):
    x = x_ref[...]
    ox_ref[...] = jnp.concatenate(
        [x[:, j * _TRBC:(j + 1) * _TRBC] for j in range(4)], axis=0).T
    y = y_ref[...]
    oy_ref[...] = jnp.concatenate(
        [y[:, j * _TRBC:(j + 1) * _TRBC] for j in range(4)], axis=0).T


def _transpose_tables(aT, bT):
    spec_in = pl.BlockSpec((D, 4 * _TRBC), lambda c: (0, c))
    spec_out = pl.BlockSpec((_TRBC, 128), lambda c: (c, 0))
    shape = jax.ShapeDtypeStruct((_VP * D // 128, 128), jnp.float32)
    return pl.pallas_call(
        _tr_body,
        grid=(_TRG,),
        in_specs=[spec_in, spec_in],
        out_specs=[spec_out, spec_out],
        out_shape=[shape, shape],
    )(aT, bT)


def _rho(ids):
    g = ids // (4 * _TRBC)
    r = ids % (4 * _TRBC)
    return g * (4 * _TRBC) + (r % _TRBC) * 4 + r // _TRBC


def kernel(center_ids, context_ids, in_embed, out_embed):
    ctxT = _rho(context_ids).T
    in_t, out_t = _transpose_tables(in_embed.T, out_embed.T)
    in_t = in_t.reshape(_VP, D)
    out_t = out_t.reshape(_VP, D)
    s, p = _sc_kernel(_rho(center_ids), ctxT, in_t, out_t)
    loss2d = pl.pallas_call(
        _tc_body,
        out_shape=jax.ShapeDtypeStruct((1, 1), jnp.float32),
    )(s.reshape(128, 128), p.reshape(128, 128))
    return loss2d[0, 0]
